# baseline (speedup 1.0000x reference)
import functools

import jax
import jax.numpy as jnp
from jax import lax
from jax.experimental import pallas as pl
from jax.experimental.pallas import tpu as pltpu
from jax.experimental.pallas import tpu_sc as plsc

N = 10000
E = 160000
B = 8
DI = 2
DU = 32
ISZ = DI + DU
SB = 40
NG = 4
F = 2 * SB
NR = NG * N

NS = 16
CH = 80
NCH = (E // NS) // CH
ROWS_PER_TILE = N // NS
ZR = 125


def _spmm_body(x_hbm, col_hbm, row_hbm, w_hbm, y_hbm,
               colb, rowb, wb, idxb, gb0, gb1, gb2, zbuf, acc,
               gs0, gs1, gs2, ss0, ss1, ss2):
    c = lax.axis_index("c")
    s = lax.axis_index("s")
    bufs = (gb0, gb1, gb2)
    gsems = (gs0, gs1, gs2)
    ssems = (ss0, ss1, ss2)

    pltpu.sync_copy(col_hbm.at[s], colb)
    pltpu.sync_copy(row_hbm.at[s], rowb)
    pltpu.sync_copy(w_hbm.at[s], wb)

    zero16 = jnp.zeros((16,), jnp.float32)

    def _zrow(i, _):
        for j in range(F // 16):
            zbuf[i, pl.ds(j * 16, 16)] = zero16
        return 0

    lax.fori_loop(0, ZR, _zrow, 0)

    def _scatter_start(k, bi):
        pltpu.async_copy(bufs[bi], acc.at[rowb.at[k]], ssems[bi], add=True)

    def _scatter_wait(k, bi):
        pltpu.make_async_copy(bufs[bi], acc.at[rowb.at[k]], ssems[bi]).wait()

    def _scale(k, bi):
        buf = bufs[bi]
        kvec = jnp.full((16,), k, jnp.int32)

        @plsc.parallel_loop(0, CH, 1, unroll=8)
        def _body(e):
            evec = jnp.full((16,), e, jnp.int32)
            wv = plsc.load_gather(wb, [kvec, evec])
            for j in range(F // 16):
                buf[e, pl.ds(j * 16, 16)] = buf[e, pl.ds(j * 16, 16)] * wv

    def _hop(src, dst, gN):
        def _gather_start(k, bi):
            pltpu.async_copy(src.at[idxb.at[k]], bufs[bi], gsems[bi])

        def _gather_wait(k, bi):
            pltpu.make_async_copy(src.at[idxb.at[k]], bufs[bi], gsems[bi]).wait()

        def _block(k, bi):
            _gather_wait(k, bi)
            _scale(k, bi)
            _scatter_start(k, bi)
            bj = (bi + 2) % 3

            @pl.when(k + 2 < NCH)
            def _():
                _scatter_wait(k - 1, bj)
                _gather_start(k + 2, bj)

        for z in range(ROWS_PER_TILE // ZR):
            pltpu.sync_copy(zbuf, acc.at[pl.ds(s * ROWS_PER_TILE + z * ZR, ZR)])

        plsc.subcore_barrier()

        _gather_start(0, 0)
        _gather_start(1, 1)
        _gather_wait(0, 0)
        _scale(0, 0)
        _scatter_start(0, 0)
        _gather_start(2, 2)

        def _trio(t, _):
            k = 3 * t
            _block(k + 1, 1)
            _block(k + 2, 2)
            _block(k + 3, 0)
            return 0

        lax.fori_loop(0, (NCH - 2) // 3, _trio, 0)
        _block(NCH - 1, (NCH - 1) % 3)
        _scatter_wait(NCH - 3, (NCH - 3) % 3)
        _scatter_wait(NCH - 2, (NCH - 2) % 3)
        _scatter_wait(NCH - 1, (NCH - 1) % 3)

        plsc.subcore_barrier()

        for z in range(ROWS_PER_TILE // ZR):
            r0 = s * ROWS_PER_TILE + z * ZR
            pltpu.sync_copy(acc.at[pl.ds(r0, ZR)], dst.at[pl.ds(gN + r0, ZR)])

        plsc.subcore_barrier()

    for p in range(2):
        gN = (2 * c + p) * N

        gvec = jnp.full((16,), gN, jnp.int32)

        def _addg(i, _):
            for j in range(CH // 16):
                idxb[i, pl.ds(j * 16, 16)] = colb[i, pl.ds(j * 16, 16)] + gvec
            return 0

        lax.fori_loop(0, NCH, _addg, 0)

        _hop(x_hbm, y_hbm, gN)


@functools.cache
def _spmm_kernel():
    mesh = plsc.VectorSubcoreMesh(core_axis_name="c", subcore_axis_name="s")
    return pl.kernel(
        _spmm_body,
        out_type=jax.ShapeDtypeStruct((NG * N, F), jnp.float32),
        mesh=mesh,
        scratch_types=[
            pltpu.VMEM((NCH, CH), jnp.int32),
            pltpu.VMEM((NCH, CH), jnp.int32),
            pltpu.VMEM((NCH, CH), jnp.float32),
            pltpu.VMEM((NCH, CH), jnp.int32),
            pltpu.VMEM((CH, F), jnp.float32),
            pltpu.VMEM((CH, F), jnp.float32),
            pltpu.VMEM((CH, F), jnp.float32),
            pltpu.VMEM((ZR, F), jnp.float32),
            pltpu.VMEM_SHARED((N, F), jnp.float32),
            pltpu.SemaphoreType.DMA,
            pltpu.SemaphoreType.DMA,
            pltpu.SemaphoreType.DMA,
            pltpu.SemaphoreType.DMA,
            pltpu.SemaphoreType.DMA,
            pltpu.SemaphoreType.DMA,
        ],
        compiler_params=pltpu.CompilerParams(use_tc_tiling_on_sc=False,
                                             needs_layout_passes=False),
    )


def _spmm2(x, colg, rowg, wg):
    y1 = _spmm_kernel()(x, colg, rowg, wg)
    y2 = _spmm_kernel()(y1, colg, rowg, wg)
    return y1, y2


RBLK = 2000
GRID = NR // RBLK


def _gate_body(x0_r, x1_r, z_r, hx_r, in_r, w_r, b_r, xp_r, u_r):
    w = w_r[...]
    g = (jnp.dot(x0_r[...], w[0:F], preferred_element_type=jnp.float32)
         + jnp.dot(x1_r[...], w[F:2 * F], preferred_element_type=jnp.float32)
         + jnp.dot(z_r[...], w[2 * F:3 * F], preferred_element_type=jnp.float32)
         + b_r[...])
    v = jax.nn.sigmoid(g)
    r_all = jnp.concatenate([v[:, 0:DU], v[:, 2 * DU:3 * DU]], axis=1)
    u_all = jnp.concatenate([v[:, DU:2 * DU], v[:, 3 * DU:4 * DU]], axis=1)
    s2 = r_all * hx_r[...]
    zpad = jnp.zeros((RBLK, F - 2 * ISZ), jnp.float32)
    xp_r[...] = jnp.concatenate([s2, in_r[...], zpad], axis=1)
    u_r[...] = u_all


def _cand_body(x0_r, x1_r, z_r, u_r, hx_r, w_r, b_r, out_r):
    w = w_r[...]
    g = (jnp.dot(x0_r[...], w[0:F], preferred_element_type=jnp.float32)
         + jnp.dot(x1_r[...], w[F:2 * F], preferred_element_type=jnp.float32)
         + jnp.dot(z_r[...], w[2 * F:3 * F], preferred_element_type=jnp.float32)
         + b_r[...])
    cand = jnp.tanh(g)
    u = u_r[...]
    out_r[...] = u * hx_r[...] + (1.0 - u) * cand


def _row_spec(width):
    return pl.BlockSpec((RBLK, width), lambda i: (i, 0))


def _full_spec(shape):
    return pl.BlockSpec(shape, lambda i: (0,) * len(shape))


def _tc_gate(x0v, x1v, zv, hxv, inv, w, b):
    return pl.pallas_call(
        _gate_body,
        grid=(GRID,),
        in_specs=[_row_spec(F), _row_spec(F), _row_spec(F),
                  _row_spec(2 * DU), _row_spec(2 * DI),
                  _full_spec((3 * F, 4 * DU)), _full_spec((1, 4 * DU))],
        out_specs=[_row_spec(F), _row_spec(2 * DU)],
        out_shape=[jax.ShapeDtypeStruct((NR, F), jnp.float32),
                   jax.ShapeDtypeStruct((NR, 2 * DU), jnp.float32)],
    )(x0v, x1v, zv, hxv, inv, w, b)


def _tc_cand(x0v, x1v, zv, uv, hxv, w, b):
    return pl.pallas_call(
        _cand_body,
        grid=(GRID,),
        in_specs=[_row_spec(F), _row_spec(F), _row_spec(F),
                  _row_spec(2 * DU), _row_spec(2 * DU),
                  _full_spec((3 * F, 2 * DU)), _full_spec((1, 2 * DU))],
        out_specs=_row_spec(2 * DU),
        out_shape=jax.ShapeDtypeStruct((NR, 2 * DU), jnp.float32),
    )(x0v, x1v, zv, uv, hxv, w, b)


NBK = 1000
PGRID = N // NBK


def _pack_body(h0_r, h1_r, i0_r, i1_r, x0_r, hxo_r, ino_r):
    h0, h1, i0, i1 = h0_r[0], h1_r[0], i0_r[0], i1_r[0]
    zpad = jnp.zeros((NBK, F - 2 * ISZ), jnp.float32)
    x0_r[...] = jnp.concatenate([h0, h1, i0, i1, zpad], axis=1)
    hxo_r[...] = jnp.concatenate([h0, h1], axis=1)
    ino_r[...] = jnp.concatenate([i0, i1], axis=1)


def _tc_pack(inputs, hx):
    hx3 = hx.reshape(B, N, DU)
    in3 = inputs.reshape(B, N, DI)
    return pl.pallas_call(
        _pack_body,
        grid=(NG, PGRID),
        in_specs=[
            pl.BlockSpec((1, NBK, DU), lambda g, i: (2 * g, i, 0)),
            pl.BlockSpec((1, NBK, DU), lambda g, i: (2 * g + 1, i, 0)),
            pl.BlockSpec((1, NBK, DI), lambda g, i: (2 * g, i, 0)),
            pl.BlockSpec((1, NBK, DI), lambda g, i: (2 * g + 1, i, 0)),
        ],
        out_specs=[
            pl.BlockSpec((NBK, F), lambda g, i: (g * PGRID + i, 0)),
            pl.BlockSpec((NBK, 2 * DU), lambda g, i: (g * PGRID + i, 0)),
            pl.BlockSpec((NBK, 2 * DI), lambda g, i: (g * PGRID + i, 0)),
        ],
        out_shape=[jax.ShapeDtypeStruct((NR, F), jnp.float32),
                   jax.ShapeDtypeStruct((NR, 2 * DU), jnp.float32),
                   jax.ShapeDtypeStruct((NR, 2 * DI), jnp.float32)],
    )(hx3, hx3, in3, in3)


def _unpack_body(new_r, out_r):
    x = new_r[...]
    b2 = pl.program_id(1)
    out_r[...] = jnp.where(b2 == 0, x[:, 0, :], x[:, 1, :])


def _tc_unpack(new_flat):
    return pl.pallas_call(
        _unpack_body,
        grid=(NG, 2, PGRID),
        in_specs=[pl.BlockSpec((NBK, 2, DU),
                               lambda g, b2, i: (g * PGRID + i, 0, 0))],
        out_specs=pl.BlockSpec((NBK, DU),
                               lambda g, b2, i: ((2 * g + b2) * PGRID + i, 0)),
        out_shape=jax.ShapeDtypeStruct((B * N, DU), jnp.float32),
    )(new_flat.reshape(NR, 2, DU)).reshape(B, N * DU)


def _prep_w(w):
    o = w.shape[1]
    w3 = w.reshape(ISZ, 3, o)
    v0 = w3[:, 0] - w3[:, 2]
    v1 = w3[:, 1]
    v2 = 2.0 * w3[:, 2]
    zh = jnp.zeros((DU, o), w.dtype)
    zi = jnp.zeros((DI, o), w.dtype)
    zp = jnp.zeros((F - 2 * ISZ, 2 * o), w.dtype)

    def blk(v):
        vh, vi = v[DI:], v[:DI]
        return jnp.concatenate([
            jnp.concatenate([vh, zh], axis=1),
            jnp.concatenate([zh, vh], axis=1),
            jnp.concatenate([vi, zi], axis=1),
            jnp.concatenate([zi, vi], axis=1),
            zp], axis=0)

    return jnp.concatenate([blk(v0), blk(v1), blk(v2)], axis=0)


def kernel(inputs, hx, edge_w, W_fn, b_fn, W_g, b_g, edge_row, edge_col):
    x0f, hx_flat, in_flat = _tc_pack(inputs, hx)

    colg = edge_col.reshape(NS, NCH, CH)
    rowg = edge_row.reshape(NS, NCH, CH)
    wg = edge_w.reshape(NS, NCH, CH)

    x1f, zf = _spmm2(x0f, colg, rowg, wg)

    wf = _prep_w(W_fn)
    wg2 = _prep_w(W_g)
    b2f = jnp.concatenate([b_fn, b_fn]).reshape(1, 4 * DU)
    b2g = jnp.concatenate([b_g, b_g]).reshape(1, 2 * DU)

    xp_flat, u_flat = _tc_gate(x0f, x1f, zf, hx_flat, in_flat, wf, b2f)

    x1p, zp = _spmm2(xp_flat, colg, rowg, wg)

    new_flat = _tc_cand(xp_flat, x1p, zp, u_flat, hx_flat, wg2, b2g)

    return _tc_unpack(new_flat)


# speedup vs baseline: 1.0877x; 1.0877x over previous
import functools

import jax
import jax.numpy as jnp
from jax import lax
from jax.experimental import pallas as pl
from jax.experimental.pallas import tpu as pltpu
from jax.experimental.pallas import tpu_sc as plsc

N = 10000
E = 160000
B = 8
DI = 2
DU = 32
ISZ = DI + DU
SB = 40
NG = 4
F = 2 * SB
NR = NG * N

NS = 16
CH = 80
NCH = (E // NS) // CH
ROWS_PER_TILE = N // NS
ZR = 125


def _spmm_body(x_hbm, col_hbm, row_hbm, w_hbm, y_hbm,
               colb, rowb, wb, idxb, gb0, gb1, gb2, zbuf, acc,
               gs0, gs1, gs2, ss0, ss1, ss2):
    c = lax.axis_index("c")
    s = lax.axis_index("s")
    bufs = (gb0, gb1, gb2)
    gsems = (gs0, gs1, gs2)
    ssems = (ss0, ss1, ss2)

    pltpu.sync_copy(col_hbm.at[s], colb)
    pltpu.sync_copy(row_hbm.at[s], rowb)
    pltpu.sync_copy(w_hbm.at[s], wb)

    zero16 = jnp.zeros((16,), jnp.float32)

    def _zrow(i, _):
        for j in range(F // 16):
            zbuf[i, pl.ds(j * 16, 16)] = zero16
        return 0

    lax.fori_loop(0, ZR, _zrow, 0)

    def _scatter_start(k, bi):
        pltpu.async_copy(bufs[bi], acc.at[rowb.at[k]], ssems[bi], add=True)

    def _scatter_wait(k, bi):
        pltpu.make_async_copy(bufs[bi], acc.at[rowb.at[k]], ssems[bi]).wait()

    def _scale(k, bi):
        buf = bufs[bi]
        kvec = jnp.full((16,), k, jnp.int32)

        @plsc.parallel_loop(0, CH, 1, unroll=8)
        def _body(e):
            evec = jnp.full((16,), e, jnp.int32)
            wv = plsc.load_gather(wb, [kvec, evec])
            for j in range(F // 16):
                buf[e, pl.ds(j * 16, 16)] = buf[e, pl.ds(j * 16, 16)] * wv

    def _hop(src, dst, gN):
        def _gather_start(k, bi):
            pltpu.async_copy(src.at[idxb.at[k]], bufs[bi], gsems[bi])

        def _gather_wait(k, bi):
            pltpu.make_async_copy(src.at[idxb.at[k]], bufs[bi], gsems[bi]).wait()

        def _block(k, bi):
            _gather_wait(k, bi)
            _scale(k, bi)
            _scatter_start(k, bi)
            bj = (bi + 2) % 3

            @pl.when(k + 2 < NCH)
            def _():
                _scatter_wait(k - 1, bj)
                _gather_start(k + 2, bj)

        for z in range(ROWS_PER_TILE // ZR):
            pltpu.sync_copy(zbuf, acc.at[pl.ds(s * ROWS_PER_TILE + z * ZR, ZR)])

        plsc.subcore_barrier()

        _gather_start(0, 0)
        _gather_start(1, 1)
        _gather_wait(0, 0)
        _scale(0, 0)
        _scatter_start(0, 0)
        _gather_start(2, 2)

        def _trio(t, _):
            k = 3 * t
            _block(k + 1, 1)
            _block(k + 2, 2)
            _block(k + 3, 0)
            return 0

        lax.fori_loop(0, (NCH - 2) // 3, _trio, 0)
        _block(NCH - 1, (NCH - 1) % 3)
        _scatter_wait(NCH - 3, (NCH - 3) % 3)
        _scatter_wait(NCH - 2, (NCH - 2) % 3)
        _scatter_wait(NCH - 1, (NCH - 1) % 3)

        plsc.subcore_barrier()

        for z in range(ROWS_PER_TILE // ZR):
            r0 = s * ROWS_PER_TILE + z * ZR
            pltpu.sync_copy(acc.at[pl.ds(r0, ZR)], dst.at[pl.ds(gN + r0, ZR)])

        plsc.subcore_barrier()

    for p in range(2):
        gN = (2 * c + p) * N

        gvec = jnp.full((16,), gN, jnp.int32)

        def _addg(i, _):
            for j in range(CH // 16):
                idxb[i, pl.ds(j * 16, 16)] = colb[i, pl.ds(j * 16, 16)] + gvec
            return 0

        lax.fori_loop(0, NCH, _addg, 0)

        _hop(x_hbm, y_hbm, gN)


@functools.cache
def _spmm_kernel():
    mesh = plsc.VectorSubcoreMesh(core_axis_name="c", subcore_axis_name="s")
    return pl.kernel(
        _spmm_body,
        out_type=jax.ShapeDtypeStruct((NG * N, F), jnp.float32),
        mesh=mesh,
        scratch_types=[
            pltpu.VMEM((NCH, CH), jnp.int32),
            pltpu.VMEM((NCH, CH), jnp.int32),
            pltpu.VMEM((NCH, CH), jnp.float32),
            pltpu.VMEM((NCH, CH), jnp.int32),
            pltpu.VMEM((CH, F), jnp.float32),
            pltpu.VMEM((CH, F), jnp.float32),
            pltpu.VMEM((CH, F), jnp.float32),
            pltpu.VMEM((ZR, F), jnp.float32),
            pltpu.VMEM_SHARED((N, F), jnp.float32),
            pltpu.SemaphoreType.DMA,
            pltpu.SemaphoreType.DMA,
            pltpu.SemaphoreType.DMA,
            pltpu.SemaphoreType.DMA,
            pltpu.SemaphoreType.DMA,
            pltpu.SemaphoreType.DMA,
        ],
        compiler_params=pltpu.CompilerParams(use_tc_tiling_on_sc=False,
                                             needs_layout_passes=False),
    )


def _spmm2(x, colg, rowg, wg):
    y1 = _spmm_kernel()(x, colg, rowg, wg)
    y2 = _spmm_kernel()(y1, colg, rowg, wg)
    return y1, y2


RBLK = 2000
GRID = NR // RBLK


def _gate_body(x0_r, x1_r, z_r, hx_r, in_r, w_r, b_r, xp_r, u_r):
    w = w_r[...]
    g = (jnp.dot(x0_r[...], w[0:F], preferred_element_type=jnp.float32)
         + jnp.dot(x1_r[...], w[F:2 * F], preferred_element_type=jnp.float32)
         + jnp.dot(z_r[...], w[2 * F:3 * F], preferred_element_type=jnp.float32)
         + b_r[...])
    v = jax.nn.sigmoid(g)
    r_all = jnp.concatenate([v[:, 0:DU], v[:, 2 * DU:3 * DU]], axis=1)
    u_all = jnp.concatenate([v[:, DU:2 * DU], v[:, 3 * DU:4 * DU]], axis=1)
    s2 = r_all * hx_r[...]
    zpad = jnp.zeros((RBLK, F - 2 * ISZ), jnp.float32)
    xp_r[...] = jnp.concatenate([s2, in_r[...], zpad], axis=1)
    u_r[...] = u_all


def _cand_body(x0_r, x1_r, z_r, u_r, hx_r, w_r, b_r, out_r):
    w = w_r[...]
    g = (jnp.dot(x0_r[...], w[0:F], preferred_element_type=jnp.float32)
         + jnp.dot(x1_r[...], w[F:2 * F], preferred_element_type=jnp.float32)
         + jnp.dot(z_r[...], w[2 * F:3 * F], preferred_element_type=jnp.float32)
         + b_r[...])
    cand = jnp.tanh(g)
    u = u_r[...]
    out_r[...] = u * hx_r[...] + (1.0 - u) * cand


def _row_spec(width):
    return pl.BlockSpec((RBLK, width), lambda i: (i, 0))


def _full_spec(shape):
    return pl.BlockSpec(shape, lambda i: (0,) * len(shape))


def _tc_gate(x0v, x1v, zv, hxv, inv, w, b):
    return pl.pallas_call(
        _gate_body,
        grid=(GRID,),
        in_specs=[_row_spec(F), _row_spec(F), _row_spec(F),
                  _row_spec(2 * DU), _row_spec(2 * DI),
                  _full_spec((3 * F, 4 * DU)), _full_spec((1, 4 * DU))],
        out_specs=[_row_spec(F), _row_spec(2 * DU)],
        out_shape=[jax.ShapeDtypeStruct((NR, F), jnp.float32),
                   jax.ShapeDtypeStruct((NR, 2 * DU), jnp.float32)],
    )(x0v, x1v, zv, hxv, inv, w, b)


def _tc_cand(x0v, x1v, zv, uv, hxv, w, b):
    return pl.pallas_call(
        _cand_body,
        grid=(GRID,),
        in_specs=[_row_spec(F), _row_spec(F), _row_spec(F),
                  _row_spec(2 * DU), _row_spec(2 * DU),
                  _full_spec((3 * F, 2 * DU)), _full_spec((1, 2 * DU))],
        out_specs=_row_spec(2 * DU),
        out_shape=jax.ShapeDtypeStruct((NR, 2 * DU), jnp.float32),
    )(x0v, x1v, zv, uv, hxv, w, b)


NBK = 1000
PGRID = N // NBK


def _pack_body(h0_r, h1_r, i0_r, i1_r, x0_r, hxo_r, ino_r):
    h0, h1, i0, i1 = h0_r[0], h1_r[0], i0_r[0], i1_r[0]
    zpad = jnp.zeros((NBK, F - 2 * ISZ), jnp.float32)
    x0_r[...] = jnp.concatenate([h0, h1, i0, i1, zpad], axis=1)
    hxo_r[...] = jnp.concatenate([h0, h1], axis=1)
    ino_r[...] = jnp.concatenate([i0, i1], axis=1)


def _tc_pack(inputs, hx):
    hx3 = hx.reshape(B, N, DU)
    in3 = inputs.reshape(B, N, DI)
    return pl.pallas_call(
        _pack_body,
        grid=(NG, PGRID),
        in_specs=[
            pl.BlockSpec((1, NBK, DU), lambda g, i: (2 * g, i, 0)),
            pl.BlockSpec((1, NBK, DU), lambda g, i: (2 * g + 1, i, 0)),
            pl.BlockSpec((1, NBK, DI), lambda g, i: (2 * g, i, 0)),
            pl.BlockSpec((1, NBK, DI), lambda g, i: (2 * g + 1, i, 0)),
        ],
        out_specs=[
            pl.BlockSpec((NBK, F), lambda g, i: (g * PGRID + i, 0)),
            pl.BlockSpec((NBK, 2 * DU), lambda g, i: (g * PGRID + i, 0)),
            pl.BlockSpec((NBK, 2 * DI), lambda g, i: (g * PGRID + i, 0)),
        ],
        out_shape=[jax.ShapeDtypeStruct((NR, F), jnp.float32),
                   jax.ShapeDtypeStruct((NR, 2 * DU), jnp.float32),
                   jax.ShapeDtypeStruct((NR, 2 * DI), jnp.float32)],
    )(hx3, hx3, in3, in3)


def _prep_w(w):
    o = w.shape[1]
    w3 = w.reshape(ISZ, 3, o)
    v0 = w3[:, 0] - w3[:, 2]
    v1 = w3[:, 1]
    v2 = 2.0 * w3[:, 2]
    zh = jnp.zeros((DU, o), w.dtype)
    zi = jnp.zeros((DI, o), w.dtype)
    zp = jnp.zeros((F - 2 * ISZ, 2 * o), w.dtype)

    def blk(v):
        vh, vi = v[DI:], v[:DI]
        return jnp.concatenate([
            jnp.concatenate([vh, zh], axis=1),
            jnp.concatenate([zh, vh], axis=1),
            jnp.concatenate([vi, zi], axis=1),
            jnp.concatenate([zi, vi], axis=1),
            zp], axis=0)

    return jnp.concatenate([blk(v0), blk(v1), blk(v2)], axis=0)


def kernel(inputs, hx, edge_w, W_fn, b_fn, W_g, b_g, edge_row, edge_col):
    x0f, hx_flat, in_flat = _tc_pack(inputs, hx)

    colg = edge_col.reshape(NS, NCH, CH)
    rowg = edge_row.reshape(NS, NCH, CH)
    wg = edge_w.reshape(NS, NCH, CH)

    x1f, zf = _spmm2(x0f, colg, rowg, wg)

    wf = _prep_w(W_fn)
    wg2 = _prep_w(W_g)
    b2f = jnp.concatenate([b_fn, b_fn]).reshape(1, 4 * DU)
    b2g = jnp.concatenate([b_g, b_g]).reshape(1, 2 * DU)

    xp_flat, u_flat = _tc_gate(x0f, x1f, zf, hx_flat, in_flat, wf, b2f)

    x1p, zp = _spmm2(xp_flat, colg, rowg, wg)

    new_flat = _tc_cand(xp_flat, x1p, zp, u_flat, hx_flat, wg2, b2g)

    return (new_flat.reshape(NG, N, 2, DU)
            .transpose(0, 2, 1, 3)
            .reshape(B, N * DU))
